# Initial kernel scaffold; baseline (speedup 1.0000x reference)
import jax
import jax.numpy as jnp
from jax.experimental import pallas as pl


def kernel(x, edge_index, edge_attr, params):
    raise NotImplementedError("write your pallas kernel here")


# speedup vs baseline: 1.5697x; 1.5697x over previous
import functools

import jax
import jax.numpy as jnp
from jax import lax
from jax.experimental import pallas as pl
from jax.experimental.pallas import tpu as pltpu
from jax.experimental.pallas import tpu_sc as plsc

HID = 128
IN_EDGE = 16
N_NODES = 10000
N_EDGES = 320000
NWORK = 32
NCHUNK = 64
NCH = 160
NPAD = NCHUNK * NCH
KE = 128
KU = 80
EPW = N_EDGES // NWORK
BIG = 3.0e38
LN_EPS = 1e-5
VAR_EPS = 1e-5


def _silu(v):
    return v * jax.nn.sigmoid(v)


def _layernorm(h):
    m = jnp.mean(h, axis=1, keepdims=True)
    var = jnp.mean((h - m) ** 2, axis=1, keepdims=True)
    return (h - m) * lax.rsqrt(var + LN_EPS)


def _linear_silu4_body(hid_ref, w1, b1, w2, b2, w3, b3, w4, b4,
                       o1, o2, o3, o4):
    hn = _layernorm(hid_ref[...])
    for w, b, o in ((w1, b1, o1), (w2, b2, o2), (w3, b3, o3), (w4, b4, o4)):
        v = jnp.dot(hn, w[...], preferred_element_type=jnp.float32) + b[...]
        o[...] = _silu(v)


def _node_tables(hidden, c):
    BN = 1000
    grid = N_NODES // BN
    row = pl.BlockSpec((BN, HID), lambda i: (i, 0))
    wsp = pl.BlockSpec((HID, HID), lambda i: (0, 0))
    bsp = pl.BlockSpec((1, HID), lambda i: (0, 0))
    outs = [jax.ShapeDtypeStruct((N_NODES, HID), jnp.float32)] * 4
    args = []
    for k in ("lin1", "lin2", "lin3", "lin4"):
        args += [c[k]["W"], c[k]["b"].reshape(1, HID)]
    return pl.pallas_call(
        _linear_silu4_body,
        grid=(grid,),
        in_specs=[row] + [wsp, bsp] * 4,
        out_specs=[row] * 4,
        out_shape=outs,
    )(hidden, *args)


def _encode_nodes_body(x_ref, w, b, o):
    o[...] = jnp.dot(x_ref[...], w[...],
                     preferred_element_type=jnp.float32) + b[...]


def _encode_nodes(x, p):
    BN = 1000
    return pl.pallas_call(
        _encode_nodes_body,
        grid=(N_NODES // BN,),
        in_specs=[pl.BlockSpec((BN, HID), lambda i: (i, 0)),
                  pl.BlockSpec((HID, HID), lambda i: (0, 0)),
                  pl.BlockSpec((1, HID), lambda i: (0, 0))],
        out_specs=pl.BlockSpec((BN, HID), lambda i: (i, 0)),
        out_shape=jax.ShapeDtypeStruct((N_NODES, HID), jnp.float32),
    )(x, p["W"], p["b"].reshape(1, HID))


def _encode_edges_body(e_ref, w, b, o):
    o[...] = jnp.dot(e_ref[...], w[...],
                     preferred_element_type=jnp.float32) + b[...]


def _encode_edges(edge_attr, p):
    BE = 8000
    return pl.pallas_call(
        _encode_edges_body,
        grid=(N_EDGES // BE,),
        in_specs=[pl.BlockSpec((BE, IN_EDGE), lambda i: (i, 0)),
                  pl.BlockSpec((IN_EDGE, HID), lambda i: (0, 0)),
                  pl.BlockSpec((1, HID), lambda i: (0, 0))],
        out_specs=pl.BlockSpec((BE, HID), lambda i: (i, 0)),
        out_shape=jax.ShapeDtypeStruct((N_EDGES, HID), jnp.float32),
    )(edge_attr, p["W"], p["b"].reshape(1, HID))


def _node_update_body(hid_ref, z_ref, cnts_ref,
                      sd, qd, xd, nd, ss, qs, xs, ns,
                      wd, bd, out):
    inv = jnp.where(
        lax.broadcasted_iota(jnp.int32, (1, HID), 1) < HID // 2, 1.0, -1.0)
    z = z_ref[...]
    w = wd[...]

    def rn(f, s, q, mx, mn, cnt):
        cc = jnp.maximum(cnt, 1.0)
        mp = s[...] / cc
        m2p = q[...] / cc
        varp = m2p - mp * mp
        mean = f * mp
        v2 = (f * f) * varp
        std = jnp.where(v2 <= VAR_EPS, 0.0, jnp.sqrt(jnp.maximum(v2, VAR_EPS)))
        has = cnt > 0
        pos = f >= 0
        mxa = jnp.where(has, jnp.where(pos, f * mx[...], f * mn[...]), 0.0)
        mna = jnp.where(has, jnp.where(pos, f * mn[...], f * mx[...]), 0.0)
        acc = jnp.dot(mean, w[0:HID], preferred_element_type=jnp.float32)
        acc += jnp.dot(std, w[HID:2 * HID], preferred_element_type=jnp.float32)
        acc += jnp.dot(mxa, w[2 * HID:3 * HID], preferred_element_type=jnp.float32)
        acc += jnp.dot(mna, w[3 * HID:4 * HID], preferred_element_type=jnp.float32)
        return acc

    cnt_d = cnts_ref[:, 0:1]
    cnt_s = cnts_ref[:, 1:2]
    out[...] = (hid_ref[...] + rn(z, sd, qd, xd, nd, cnt_d)
                + rn(z * inv, ss, qs, xs, ns, cnt_s) + 2.0 * bd[...])


def _node_update(hidden, z, cnts, stats_d, stats_s, pdown):
    BN = 1000
    row = pl.BlockSpec((BN, HID), lambda i: (i, 0))
    return pl.pallas_call(
        _node_update_body,
        grid=(N_NODES // BN,),
        in_specs=[row, row,
                  pl.BlockSpec((BN, 8), lambda i: (i, 0))]
                 + [row] * 8
                 + [pl.BlockSpec((4 * HID, HID), lambda i: (0, 0)),
                    pl.BlockSpec((1, HID), lambda i: (0, 0))],
        out_specs=row,
        out_shape=jax.ShapeDtypeStruct((N_NODES, HID), jnp.float32),
    )(hidden, z, cnts, *stats_d, *stats_s,
      pdown["W"], pdown["b"].reshape(1, HID))


def _node_decode_body(x_ref, hid_ref, w1, b1, w2, b2, out):
    hn = _layernorm(hid_ref[...])
    v = _silu(jnp.dot(hn, w1[...], preferred_element_type=jnp.float32) + b1[...])
    d = jnp.dot(v, w2[...], preferred_element_type=jnp.float32) + b2[...]
    out[...] = x_ref[...] + 0.01 * d


def _node_decode(x, hidden, p):
    BN = 1000
    row = pl.BlockSpec((BN, HID), lambda i: (i, 0))
    wsp = pl.BlockSpec((HID, HID), lambda i: (0, 0))
    bsp = pl.BlockSpec((1, HID), lambda i: (0, 0))
    return pl.pallas_call(
        _node_decode_body,
        grid=(N_NODES // BN,),
        in_specs=[row, row, wsp, bsp, wsp, bsp],
        out_specs=row,
        out_shape=jax.ShapeDtypeStruct((N_NODES, HID), jnp.float32),
    )(x, hidden, p["l1"]["W"], p["l1"]["b"].reshape(1, HID),
      p["l2"]["W"], p["l2"]["b"].reshape(1, HID))


def _edge_decode_body(e_ref, ea_ref, w1, b1, w2, b2, out):
    hn = _layernorm(ea_ref[...])
    v = _silu(jnp.dot(hn, w1[...], preferred_element_type=jnp.float32) + b1[...])
    d = jnp.dot(v, w2[...], preferred_element_type=jnp.float32) + b2[...]
    out[...] = e_ref[...] + 0.01 * d


def _edge_decode(edge_attr, ea, p):
    BE = 4000
    return pl.pallas_call(
        _edge_decode_body,
        grid=(N_EDGES // BE,),
        in_specs=[pl.BlockSpec((BE, IN_EDGE), lambda i: (i, 0)),
                  pl.BlockSpec((BE, HID), lambda i: (i, 0)),
                  pl.BlockSpec((HID, HID), lambda i: (0, 0)),
                  pl.BlockSpec((1, HID), lambda i: (0, 0)),
                  pl.BlockSpec((HID, IN_EDGE), lambda i: (0, 0)),
                  pl.BlockSpec((1, IN_EDGE), lambda i: (0, 0))],
        out_specs=pl.BlockSpec((BE, IN_EDGE), lambda i: (i, 0)),
        out_shape=jax.ShapeDtypeStruct((N_EDGES, IN_EDGE), jnp.float32),
    )(edge_attr, ea, p["l1"]["W"], p["l1"]["b"].reshape(1, HID),
      p["l2"]["W"], p["l2"]["b"].reshape(1, IN_EDGE))


def _seg_stats(tab, ea, oth, prm, seg, bounds):
    mesh = plsc.VectorSubcoreMesh(core_axis_name="c", subcore_axis_name="s")
    sds = jax.ShapeDtypeStruct((NPAD, HID), jnp.float32)

    @functools.partial(
        pl.kernel,
        mesh=mesh,
        out_type=(sds, sds, sds, sds),
        scratch_types=[
            pltpu.VMEM((NCH + 1, HID), jnp.float32),
            pltpu.VMEM((NCH + 1, HID), jnp.float32),
            pltpu.VMEM((NCH + 1, HID), jnp.float32),
            pltpu.VMEM((NCH + 1, HID), jnp.float32),
            pltpu.VMEM((KE, HID), jnp.float32),
            pltpu.VMEM((KE, HID), jnp.float32),
            pltpu.VMEM((KE + 16,), jnp.int32),
            pltpu.VMEM((KE + 16,), jnp.int32),
            pltpu.VMEM((KE + 16,), jnp.int32),
            pltpu.VMEM((NCHUNK + 16,), jnp.int32),
            pltpu.SemaphoreType.DMA,
            pltpu.SemaphoreType.DMA,
        ],
    )
    def k(tab_h, ea_h, oth_h, prm_h, seg_h, bnd_h,
          osum, osq, omx, omn,
          asum, asq, amx, amn, hbuf, ebuf, othv, prmv, segv, bnd,
          sem1, sem2):
        wid = lax.axis_index("s") * 2 + lax.axis_index("c")
        pltpu.sync_copy(bnd_h, bnd.at[pl.ds(0, NCHUNK + 1)])
        zero = jnp.zeros((16,), jnp.float32)
        neg = jnp.full((16,), -BIG, jnp.float32)
        pos = jnp.full((16,), BIG, jnp.float32)

        for rep in range(2):
            chunk = wid * 2 + rep
            base_node = chunk * NCH

            @pl.loop(0, (NCH + 1) * 8)
            def zinit(i):
                r = i // 8
                c = lax.rem(i, 8) * 16
                asum[r, pl.ds(c, 16)] = zero
                asq[r, pl.ds(c, 16)] = zero
                amx[r, pl.ds(c, 16)] = neg
                amn[r, pl.ds(c, 16)] = pos

            bv = bnd[pl.ds(chunk, 16)]
            lo = bv[0]
            hi = bv[1]
            e0f = lo - lax.rem(lo, 8)
            ntrip = (hi - e0f + (KE - 1)) // KE

            @pl.loop(0, ntrip)
            def blk(t):
                e0 = pl.multiple_of(e0f + t * KE, 8)
                pltpu.sync_copy(oth_h.at[pl.ds(e0, KE)], othv.at[pl.ds(0, KE)])
                pltpu.sync_copy(prm_h.at[pl.ds(e0, KE)], prmv.at[pl.ds(0, KE)])
                pltpu.sync_copy(seg_h.at[pl.ds(e0, KE)], segv.at[pl.ds(0, KE)])
                for i in range(KE // 16):
                    sl = pl.ds(i * 16, 16)
                    ev = e0 + i * 16 + lax.iota(jnp.int32, 16)
                    valid = (ev >= lo) & (ev < hi)
                    othv[sl] = jnp.where(valid, othv[sl], 0)
                    prmv[sl] = jnp.where(valid, prmv[sl], 0)
                    segv[sl] = jnp.where(valid, segv[sl] - base_node, NCH)
                cp1 = pltpu.async_copy(tab_h.at[othv.at[pl.ds(0, KE)]],
                                       hbuf, sem1)
                cp2 = pltpu.async_copy(ea_h.at[prmv.at[pl.ds(0, KE)]],
                                       ebuf, sem2)
                cp1.wait()
                cp2.wait()

                @pl.loop(0, KE)
                def edge(j):
                    r = segv[pl.ds(j, 16)][0]
                    for c in range(HID // 16):
                        sl = pl.ds(c * 16, 16)
                        p = hbuf[j, sl] * ebuf[j, sl]
                        plsc.addupdate(asum.at[r, sl], p)
                        plsc.addupdate(asq.at[r, sl], p * p)
                        amx[r, sl] = jnp.maximum(amx[r, sl], p)
                        amn[r, sl] = jnp.minimum(amn[r, sl], p)

            pltpu.sync_copy(asum.at[pl.ds(0, NCH)],
                            osum.at[pl.ds(base_node, NCH)])
            pltpu.sync_copy(asq.at[pl.ds(0, NCH)],
                            osq.at[pl.ds(base_node, NCH)])
            pltpu.sync_copy(amx.at[pl.ds(0, NCH)],
                            omx.at[pl.ds(base_node, NCH)])
            pltpu.sync_copy(amn.at[pl.ds(0, NCH)],
                            omn.at[pl.ds(base_node, NCH)])

    return k(tab, ea, oth, prm, seg, bounds)


def _edge_update(h3, h4, ea, src, dst):
    mesh = plsc.VectorSubcoreMesh(core_axis_name="c", subcore_axis_name="s")

    @functools.partial(
        pl.kernel,
        mesh=mesh,
        out_type=jax.ShapeDtypeStruct((N_EDGES, HID), jnp.float32),
        scratch_types=[
            pltpu.VMEM((KU, HID), jnp.float32),
            pltpu.VMEM((KU, HID), jnp.float32),
            pltpu.VMEM((KU, HID), jnp.float32),
            pltpu.VMEM((KU, HID), jnp.float32),
            pltpu.VMEM((KU, HID), jnp.float32),
            pltpu.VMEM((KU,), jnp.int32),
            pltpu.VMEM((KU,), jnp.int32),
            pltpu.SemaphoreType.DMA,
            pltpu.SemaphoreType.DMA,
            pltpu.SemaphoreType.DMA,
            pltpu.SemaphoreType.DMA,
        ],
    )
    def k(h3_h, h4_h, ea_h, src_h, dst_h, out_h,
          ebuf, g1, g2, g3, g4, sv, dv, s1, s2, s3, s4):
        wid = lax.axis_index("s") * 2 + lax.axis_index("c")
        base = wid * EPW

        @pl.loop(0, EPW // KU)
        def blk(t):
            e0 = pl.multiple_of(base + t * KU, 8)
            pltpu.sync_copy(src_h.at[pl.ds(e0, KU)], sv)
            pltpu.sync_copy(dst_h.at[pl.ds(e0, KU)], dv)
            pltpu.sync_copy(ea_h.at[pl.ds(e0, KU)], ebuf)
            cp1 = pltpu.async_copy(h3_h.at[sv], g1, s1)
            cp2 = pltpu.async_copy(h4_h.at[dv], g2, s2)
            cp3 = pltpu.async_copy(h3_h.at[dv], g3, s3)
            cp4 = pltpu.async_copy(h4_h.at[sv], g4, s4)
            cp1.wait()
            cp2.wait()
            cp3.wait()
            cp4.wait()

            @pl.loop(0, KU)
            def row(j):
                for c in range(HID // 16):
                    sl = pl.ds(c * 16, 16)
                    e = ebuf[j, sl]
                    ebuf[j, sl] = e * (1.0 + g1[j, sl] * g2[j, sl]
                                       + g3[j, sl] * g4[j, sl])

            pltpu.sync_copy(ebuf, out_h.at[pl.ds(e0, KU)])

    return k(h3, h4, ea, src, dst)


def kernel(x, edge_index, edge_attr, params):
    p = params
    src = edge_index[0]
    dst = edge_index[1]

    perm_d = jnp.argsort(dst).astype(jnp.int32)
    perm_s = jnp.argsort(src).astype(jnp.int32)
    dst_sorted = dst[perm_d]
    src_sorted = src[perm_s]
    oth_d = src[perm_d]
    oth_s = dst[perm_s]

    ar = jnp.arange(N_NODES + 1, dtype=jnp.int32)
    lo_d = jnp.searchsorted(dst_sorted, ar).astype(jnp.int32)
    lo_s = jnp.searchsorted(src_sorted, ar).astype(jnp.int32)
    cnt_d = (lo_d[1:] - lo_d[:-1]).astype(jnp.float32)
    cnt_s = (lo_s[1:] - lo_s[:-1]).astype(jnp.float32)
    cnts = jnp.concatenate(
        [cnt_d[:, None], cnt_s[:, None], jnp.zeros((N_NODES, 6), jnp.float32)],
        axis=1)

    chunkpos = jnp.minimum(
        jnp.arange(NCHUNK + 1, dtype=jnp.int32) * NCH, N_NODES)
    bounds_d = lo_d[chunkpos]
    bounds_s = lo_s[chunkpos]

    padi = jnp.zeros((KE,), jnp.int32)
    seg_d = jnp.concatenate([dst_sorted, padi])
    seg_s = jnp.concatenate([src_sorted, padi])
    othp_d = jnp.concatenate([oth_d, padi])
    othp_s = jnp.concatenate([oth_s, padi])
    prm_d = jnp.concatenate([perm_d, padi])
    prm_s = jnp.concatenate([perm_s, padi])

    hidden = _encode_nodes(x, p["nodeEnc"])
    ea = _encode_edges(edge_attr, p["edgeEnc"])

    for i in range(3):
        c = p["convs"][i]
        h1, z, h3, h4 = _node_tables(hidden, c)
        stats_d = _seg_stats(h1, ea, othp_d, prm_d, seg_d, bounds_d)
        stats_s = _seg_stats(h1, ea, othp_s, prm_s, seg_s, bounds_s)
        ea = _edge_update(h3, h4, ea, src, dst)
        hidden = _node_update(hidden, z, cnts, stats_d, stats_s, c["downlin"])

    node_pred = _node_decode(x, hidden, p["nodeDec"])
    edge_pred = _edge_decode(edge_attr, ea, p["edgeDec"])
    return (node_pred, edge_pred)
